# Initial kernel scaffold; baseline (speedup 1.0000x reference)
import jax
import jax.numpy as jnp
from jax.experimental import pallas as pl


def kernel(x, edge_index, W1, att_src1, att_dst1, bias1, W2, att_src2, att_dst2, bias2):
    raise NotImplementedError("write your pallas kernel here")


# speedup vs baseline: 25.8980x; 25.8980x over previous
import functools

import jax
import jax.numpy as jnp
from jax import lax
from jax.experimental import pallas as pl
from jax.experimental.pallas import tpu as pltpu
from jax.experimental.pallas import tpu_sc as plsc

N_NODES = 10000
NPAD = 10240
D_IN = 128
D_HID = 128
D_OUT = 64
N_EDGES = 320000
ETOT = N_EDGES + N_NODES
NW = 32
CHUNK = 128
CPW = 81
EPW = CPW * CHUNK
EPAD = EPW * NW
NT = 16
RPT = NPAD // NT

_f32 = jnp.float32
_i32 = jnp.int32


def _make_edge_pass(D):
    mesh = plsc.VectorSubcoreMesh(core_axis_name="c", subcore_axis_name="s",
                                  num_cores=2, num_subcores=NT)

    @functools.partial(
        pl.kernel,
        out_type=(jax.ShapeDtypeStruct((2, NPAD, D), _f32),
                  jax.ShapeDtypeStruct((2, NPAD), _f32)),
        mesh=mesh,
        compiler_params=pltpu.CompilerParams(needs_layout_passes=False),
        scratch_types=[
            pltpu.VMEM_SHARED((NPAD, D), _f32),
            pltpu.VMEM_SHARED((NPAD,), _f32),
            pltpu.VMEM((CHUNK,), _i32),
            pltpu.VMEM((CHUNK,), _i32),
            pltpu.VMEM((NPAD,), _f32),
            pltpu.VMEM((NPAD,), _f32),
            pltpu.VMEM((CHUNK,), _f32),
            pltpu.VMEM((CHUNK, D), _f32),
            pltpu.VMEM((RPT,), _f32),
            pltpu.SemaphoreType.DMA,
        ],
    )
    def edge_pass(src_hbm, dst_hbm, asrc_hbm, adst_hbm, h_hbm,
                  acc_out, den_out,
                  acc_sh, den_sh, srcv, dstv, asrc_v, adst_v, wv, rows, zvec,
                  sem):
        c = lax.axis_index("c")
        s = lax.axis_index("s")
        wid = c * NT + s
        base = s * RPT

        def zrow(r, carry):
            for k in range(D // 16):
                rows[r, pl.ds(k * 16, 16)] = jnp.zeros((16,), _f32)
            return carry
        lax.fori_loop(0, CHUNK, zrow, 0)

        def zv(m, carry):
            zvec[pl.ds(m * 16, 16)] = jnp.zeros((16,), _f32)
            return carry
        lax.fori_loop(0, RPT // 16, zv, 0)

        for t in range(RPT // CHUNK):
            pltpu.sync_copy(rows, acc_sh.at[pl.ds(base + t * CHUNK, CHUNK)])
        pltpu.sync_copy(zvec, den_sh.at[pl.ds(base, RPT)])

        pltpu.sync_copy(asrc_hbm, asrc_v)
        pltpu.sync_copy(adst_hbm, adst_v)
        plsc.subcore_barrier()

        def chunk_body(i, carry):
            pltpu.sync_copy(src_hbm.at[wid].at[i], srcv)
            pltpu.sync_copy(dst_hbm.at[wid].at[i], dstv)
            for j in range(CHUNK // 16):
                sv = srcv[pl.ds(j * 16, 16)]
                dv = dstv[pl.ds(j * 16, 16)]
                e = plsc.load_gather(asrc_v, [sv]) + plsc.load_gather(adst_v, [dv])
                e = jnp.where(e >= 0.0, e, e * 0.2)
                wv[pl.ds(j * 16, 16)] = jnp.exp(e)
            pltpu.sync_copy(wv, den_sh.at[dstv], add=True)
            pltpu.async_copy(h_hbm.at[srcv], rows, sem).wait()

            def row_body(r, rcarry):
                wb = plsc.load_gather(wv, [jnp.broadcast_to(r, (16,))])
                for k in range(D // 16):
                    rows[r, pl.ds(k * 16, 16)] = rows[r, pl.ds(k * 16, 16)] * wb
                return rcarry
            lax.fori_loop(0, CHUNK, row_body, 0)

            pltpu.sync_copy(rows, acc_sh.at[dstv], add=True)
            return carry
        lax.fori_loop(0, CPW, chunk_body, 0)

        plsc.subcore_barrier()
        pltpu.sync_copy(acc_sh.at[pl.ds(base, RPT)],
                        acc_out.at[c].at[pl.ds(base, RPT)])
        pltpu.sync_copy(den_sh.at[pl.ds(base, RPT)],
                        den_out.at[c].at[pl.ds(base, RPT)])

    return edge_pass


_edge_pass = _make_edge_pass(D_HID)


_BLK = 1024
_GRID = NPAD // _BLK


def _dense1_body(x_ref, w_ref, asc_ref, adc_ref, h_ref, as_ref, ad_ref):
    h = jnp.dot(x_ref[...], w_ref[...], preferred_element_type=_f32)
    h_ref[...] = h
    as_ref[...] = jnp.dot(h, asc_ref[...], preferred_element_type=_f32)
    ad_ref[...] = jnp.dot(h, adc_ref[...], preferred_element_type=_f32)


def _dense1(x_pad, W1, asc, adc):
    return pl.pallas_call(
        _dense1_body,
        grid=(_GRID,),
        in_specs=[
            pl.BlockSpec((_BLK, D_IN), lambda i: (i, 0)),
            pl.BlockSpec((D_IN, D_HID), lambda i: (0, 0)),
            pl.BlockSpec((D_HID, 1), lambda i: (0, 0)),
            pl.BlockSpec((D_HID, 1), lambda i: (0, 0)),
        ],
        out_specs=[
            pl.BlockSpec((_BLK, D_HID), lambda i: (i, 0)),
            pl.BlockSpec((_BLK, 1), lambda i: (i, 0)),
            pl.BlockSpec((_BLK, 1), lambda i: (i, 0)),
        ],
        out_shape=[
            jax.ShapeDtypeStruct((NPAD, D_HID), _f32),
            jax.ShapeDtypeStruct((NPAD, 1), _f32),
            jax.ShapeDtypeStruct((NPAD, 1), _f32),
        ],
    )(x_pad, W1, asc, adc)


def _dense2_body(acc_ref, den_ref, b_ref, w_ref, asc_ref, adc_ref,
                 h_ref, as_ref, ad_ref):
    den = den_ref[0] + den_ref[1]
    x2 = (acc_ref[0] + acc_ref[1]) / (den[:, None] + 1e-16) + b_ref[...]
    x2 = jnp.maximum(x2, 0.0)
    h2 = jnp.dot(x2, w_ref[...], preferred_element_type=_f32)
    h_ref[...] = h2
    as_ref[...] = jnp.dot(h2, asc_ref[...], preferred_element_type=_f32)
    ad_ref[...] = jnp.dot(h2, adc_ref[...], preferred_element_type=_f32)


def _dense2(acc, den, b1, W2, asc, adc):
    return pl.pallas_call(
        _dense2_body,
        grid=(_GRID,),
        in_specs=[
            pl.BlockSpec((2, _BLK, D_HID), lambda i: (0, i, 0)),
            pl.BlockSpec((2, _BLK), lambda i: (0, i)),
            pl.BlockSpec((1, D_HID), lambda i: (0, 0)),
            pl.BlockSpec((D_HID, D_HID), lambda i: (0, 0)),
            pl.BlockSpec((D_HID, 1), lambda i: (0, 0)),
            pl.BlockSpec((D_HID, 1), lambda i: (0, 0)),
        ],
        out_specs=[
            pl.BlockSpec((_BLK, D_HID), lambda i: (i, 0)),
            pl.BlockSpec((_BLK, 1), lambda i: (i, 0)),
            pl.BlockSpec((_BLK, 1), lambda i: (i, 0)),
        ],
        out_shape=[
            jax.ShapeDtypeStruct((NPAD, D_HID), _f32),
            jax.ShapeDtypeStruct((NPAD, 1), _f32),
            jax.ShapeDtypeStruct((NPAD, 1), _f32),
        ],
    )(acc, den, b1, W2, asc, adc)


def _final_body(acc_ref, den_ref, b_ref, out_ref):
    den = den_ref[0] + den_ref[1]
    o = (acc_ref[0, :, :D_OUT] + acc_ref[1, :, :D_OUT]) / (den[:, None] + 1e-16)
    o = o + b_ref[...]
    m = jnp.max(o, axis=1, keepdims=True)
    lse = jnp.log(jnp.sum(jnp.exp(o - m), axis=1, keepdims=True)) + m
    out_ref[...] = o - lse


def _final(acc, den, b2):
    return pl.pallas_call(
        _final_body,
        grid=(_GRID,),
        in_specs=[
            pl.BlockSpec((2, _BLK, D_HID), lambda i: (0, i, 0)),
            pl.BlockSpec((2, _BLK), lambda i: (0, i)),
            pl.BlockSpec((1, D_OUT), lambda i: (0, 0)),
        ],
        out_specs=pl.BlockSpec((_BLK, D_OUT), lambda i: (i, 0)),
        out_shape=jax.ShapeDtypeStruct((NPAD, D_OUT), _f32),
    )(acc, den, b2)


def kernel(x, edge_index, W1, att_src1, att_dst1, bias1,
           W2, att_src2, att_dst2, bias2):
    loop = jnp.arange(N_NODES, dtype=_i32)
    src = jnp.concatenate([edge_index[0].astype(_i32), loop])
    dst = jnp.concatenate([edge_index[1].astype(_i32), loop])
    npad_e = EPAD - ETOT
    pad = jnp.arange(npad_e, dtype=_i32)
    src = jnp.concatenate([src, pad % N_NODES])
    dst = jnp.concatenate([dst, N_NODES + pad % (NPAD - N_NODES)])
    src3 = src.reshape(NW, CPW, CHUNK)
    dst3 = dst.reshape(NW, CPW, CHUNK)

    x_pad = jnp.pad(x, ((0, NPAD - N_NODES), (0, 0)))

    W2p = jnp.pad(W2, ((0, 0), (0, D_HID - D_OUT)))
    as2p = jnp.pad(att_src2, (0, D_HID - D_OUT)).reshape(D_HID, 1)
    ad2p = jnp.pad(att_dst2, (0, D_HID - D_OUT)).reshape(D_HID, 1)

    h1, as1, ad1 = _dense1(x_pad, W1,
                           att_src1.reshape(D_HID, 1),
                           att_dst1.reshape(D_HID, 1))
    acc1, den1 = _edge_pass(src3, dst3,
                            as1.reshape(NPAD), ad1.reshape(NPAD), h1)
    h2, as2, ad2 = _dense2(acc1, den1, bias1.reshape(1, D_HID), W2p,
                           as2p, ad2p)
    acc2, den2 = _edge_pass(src3, dst3,
                            as2.reshape(NPAD), ad2.reshape(NPAD), h2)
    out = _final(acc2, den2, bias2.reshape(1, D_OUT))
    return out[:N_NODES]


# speedup vs baseline: 48.5572x; 1.8749x over previous
import functools

import jax
import jax.numpy as jnp
from jax import lax
from jax.experimental import pallas as pl
from jax.experimental.pallas import tpu as pltpu
from jax.experimental.pallas import tpu_sc as plsc

N_NODES = 10000
NPAD = 10240
D_IN = 128
D_HID = 128
D_OUT = 64
N_EDGES = 320000
ETOT = N_EDGES + N_NODES
NW = 32
CHUNK = 128
CPW = 82
EPW = CPW * CHUNK
EPAD = EPW * NW
NT = 16
RPT = NPAD // NT

_f32 = jnp.float32
_i32 = jnp.int32


def _make_edge_pass(mult_width):
    D = D_HID
    mesh = plsc.VectorSubcoreMesh(core_axis_name="c", subcore_axis_name="s",
                                  num_cores=2, num_subcores=NT)

    @functools.partial(
        pl.kernel,
        out_type=(jax.ShapeDtypeStruct((2, NPAD, D), _f32),
                  jax.ShapeDtypeStruct((2, NPAD), _f32)),
        mesh=mesh,
        compiler_params=pltpu.CompilerParams(needs_layout_passes=False),
        scratch_types=[
            pltpu.VMEM_SHARED((NPAD, D), _f32),
            pltpu.VMEM_SHARED((NPAD,), _f32),
            pltpu.VMEM_SHARED((NPAD,), _f32),
            pltpu.VMEM_SHARED((NPAD,), _f32),
            [pltpu.VMEM((2, CHUNK), _i32)] * 2,
            [pltpu.VMEM((1, CHUNK), _i32)] * 2,
            [pltpu.VMEM((CHUNK,), _f32)] * 2,
            [pltpu.VMEM((CHUNK, D), _f32)] * 2,
            pltpu.VMEM((CHUNK,), _f32),
            pltpu.VMEM((CHUNK,), _f32),
            pltpu.VMEM((RPT,), _f32),
            [pltpu.SemaphoreType.DMA] * 2,
            [pltpu.SemaphoreType.DMA] * 2,
            [pltpu.SemaphoreType.DMA] * 2,
            [pltpu.SemaphoreType.DMA] * 2,
        ],
    )
    def edge_pass(edges_hbm, asrc_hbm, adst_hbm, h_hbm,
                  acc_out, den_out,
                  acc_sh, den_sh, asrc_sh, adst_sh,
                  idx, dstS, wv, rows, asg, adg, zvec,
                  isem, gsem, ssem, wsem):
        c = lax.axis_index("c")
        s = lax.axis_index("s")
        wid = c * NT + s
        base = s * RPT

        def zrow(r, carry):
            for k in range(D // 16):
                rows[0][r, pl.ds(k * 16, 16)] = jnp.zeros((16,), _f32)
            return carry
        lax.fori_loop(0, CHUNK, zrow, 0)

        def zv(m, carry):
            zvec[pl.ds(m * 16, 16)] = jnp.zeros((16,), _f32)
            return carry
        lax.fori_loop(0, RPT // 16, zv, 0)

        for t in range(RPT // CHUNK):
            pltpu.sync_copy(rows[0], acc_sh.at[pl.ds(base + t * CHUNK, CHUNK)])
        pltpu.sync_copy(zvec, den_sh.at[pl.ds(base, RPT)])

        @pl.when(s == 0)
        def _():
            pltpu.sync_copy(asrc_hbm, asrc_sh)

        @pl.when(s == 1)
        def _():
            pltpu.sync_copy(adst_hbm, adst_sh)

        pltpu.async_copy(edges_hbm.at[wid].at[0], idx[0], isem[0])
        plsc.subcore_barrier()

        def do_multiply_scatter(b1):
            def row_body(r, rcarry):
                wb = plsc.load_gather(wv[b1], [jnp.broadcast_to(r, (16,))])
                for k in range(mult_width // 16):
                    rows[b1][r, pl.ds(k * 16, 16)] = (
                        rows[b1][r, pl.ds(k * 16, 16)] * wb)
                return rcarry
            lax.fori_loop(0, CHUNK, row_body, 0)

            pltpu.async_copy(rows[b1], acc_sh.at[dstS[b1].at[0]], ssem[b1],
                             add=True)

        def body(t, b):
            b1 = 1 - b
            @pl.when(t >= 1)
            def _():
                pltpu.make_async_copy(wv[b1], den_sh.at[idx[b1].at[1]],
                                      wsem[b1]).wait()
                pltpu.make_async_copy(h_hbm.at[idx[b1].at[0]], rows[b1],
                                      gsem[b1]).wait()

            @pl.when(t + 1 < CPW)
            def _():
                pltpu.async_copy(edges_hbm.at[wid].at[t + 1], idx[b1],
                                 isem[b1])

            pltpu.make_async_copy(edges_hbm.at[wid].at[t], idx[b],
                                  isem[b]).wait()
            pltpu.sync_copy(asrc_sh.at[idx[b].at[0]], asg)
            pltpu.sync_copy(adst_sh.at[idx[b].at[1]], adg)
            for j in range(CHUNK // 16):
                e = asg[pl.ds(j * 16, 16)] + adg[pl.ds(j * 16, 16)]
                e = jnp.where(e >= 0.0, e, e * 0.2)
                wv[b][pl.ds(j * 16, 16)] = jnp.exp(e)
                dstS[b][0, pl.ds(j * 16, 16)] = idx[b][1, pl.ds(j * 16, 16)]
            pltpu.async_copy(wv[b], den_sh.at[idx[b].at[1]], wsem[b],
                             add=True)

            @pl.when(t >= 2)
            def _():
                pltpu.make_async_copy(rows[b], acc_sh.at[dstS[b].at[0]],
                                      ssem[b]).wait()
            pltpu.async_copy(h_hbm.at[idx[b].at[0]], rows[b], gsem[b])

            @pl.when(t >= 1)
            def _():
                do_multiply_scatter(b1)

        def gbody(g, carry):
            body(2 * g, 0)
            body(2 * g + 1, 1)
            return carry
        lax.fori_loop(0, CPW // 2, gbody, 0)

        pltpu.make_async_copy(h_hbm.at[idx[1].at[0]], rows[1], gsem[1]).wait()
        do_multiply_scatter(1)
        pltpu.make_async_copy(wv[1], den_sh.at[idx[1].at[1]], wsem[1]).wait()
        pltpu.make_async_copy(rows[0], acc_sh.at[dstS[0].at[0]],
                              ssem[0]).wait()
        pltpu.make_async_copy(rows[1], acc_sh.at[dstS[1].at[0]],
                              ssem[1]).wait()

        plsc.subcore_barrier()
        pltpu.sync_copy(acc_sh.at[pl.ds(base, RPT)],
                        acc_out.at[c].at[pl.ds(base, RPT)])
        pltpu.sync_copy(den_sh.at[pl.ds(base, RPT)],
                        den_out.at[c].at[pl.ds(base, RPT)])

    return edge_pass


_edge_pass1 = _make_edge_pass(D_HID)
_edge_pass2 = _make_edge_pass(D_OUT)


_BLK = 1024
_GRID = NPAD // _BLK


def _dense1_body(x_ref, w_ref, asc_ref, adc_ref, h_ref, as_ref, ad_ref):
    h = jnp.dot(x_ref[...], w_ref[...], preferred_element_type=_f32)
    h_ref[...] = h
    as_ref[...] = jnp.dot(h, asc_ref[...], preferred_element_type=_f32)
    ad_ref[...] = jnp.dot(h, adc_ref[...], preferred_element_type=_f32)


def _dense1(x_pad, W1, asc, adc):
    return pl.pallas_call(
        _dense1_body,
        grid=(_GRID,),
        in_specs=[
            pl.BlockSpec((_BLK, D_IN), lambda i: (i, 0)),
            pl.BlockSpec((D_IN, D_HID), lambda i: (0, 0)),
            pl.BlockSpec((D_HID, 1), lambda i: (0, 0)),
            pl.BlockSpec((D_HID, 1), lambda i: (0, 0)),
        ],
        out_specs=[
            pl.BlockSpec((_BLK, D_HID), lambda i: (i, 0)),
            pl.BlockSpec((_BLK, 1), lambda i: (i, 0)),
            pl.BlockSpec((_BLK, 1), lambda i: (i, 0)),
        ],
        out_shape=[
            jax.ShapeDtypeStruct((NPAD, D_HID), _f32),
            jax.ShapeDtypeStruct((NPAD, 1), _f32),
            jax.ShapeDtypeStruct((NPAD, 1), _f32),
        ],
    )(x_pad, W1, asc, adc)


def _dense2_body(acc_ref, den_ref, b_ref, w_ref, asc_ref, adc_ref,
                 h_ref, as_ref, ad_ref):
    den = den_ref[0] + den_ref[1]
    x2 = (acc_ref[0] + acc_ref[1]) / (den[:, None] + 1e-16) + b_ref[...]
    x2 = jnp.maximum(x2, 0.0)
    h2 = jnp.dot(x2, w_ref[...], preferred_element_type=_f32)
    h_ref[...] = h2
    as_ref[...] = jnp.dot(h2, asc_ref[...], preferred_element_type=_f32)
    ad_ref[...] = jnp.dot(h2, adc_ref[...], preferred_element_type=_f32)


def _dense2(acc, den, b1, W2, asc, adc):
    return pl.pallas_call(
        _dense2_body,
        grid=(_GRID,),
        in_specs=[
            pl.BlockSpec((2, _BLK, D_HID), lambda i: (0, i, 0)),
            pl.BlockSpec((2, _BLK), lambda i: (0, i)),
            pl.BlockSpec((1, D_HID), lambda i: (0, 0)),
            pl.BlockSpec((D_HID, D_HID), lambda i: (0, 0)),
            pl.BlockSpec((D_HID, 1), lambda i: (0, 0)),
            pl.BlockSpec((D_HID, 1), lambda i: (0, 0)),
        ],
        out_specs=[
            pl.BlockSpec((_BLK, D_HID), lambda i: (i, 0)),
            pl.BlockSpec((_BLK, 1), lambda i: (i, 0)),
            pl.BlockSpec((_BLK, 1), lambda i: (i, 0)),
        ],
        out_shape=[
            jax.ShapeDtypeStruct((NPAD, D_HID), _f32),
            jax.ShapeDtypeStruct((NPAD, 1), _f32),
            jax.ShapeDtypeStruct((NPAD, 1), _f32),
        ],
    )(acc, den, b1, W2, asc, adc)


def _final_body(acc_ref, den_ref, b_ref, out_ref):
    den = den_ref[0] + den_ref[1]
    o = (acc_ref[0, :, :D_OUT] + acc_ref[1, :, :D_OUT]) / (den[:, None] + 1e-16)
    o = o + b_ref[...]
    m = jnp.max(o, axis=1, keepdims=True)
    lse = jnp.log(jnp.sum(jnp.exp(o - m), axis=1, keepdims=True)) + m
    out_ref[...] = o - lse


def _final(acc, den, b2):
    return pl.pallas_call(
        _final_body,
        grid=(_GRID,),
        in_specs=[
            pl.BlockSpec((2, _BLK, D_HID), lambda i: (0, i, 0)),
            pl.BlockSpec((2, _BLK), lambda i: (0, i)),
            pl.BlockSpec((1, D_OUT), lambda i: (0, 0)),
        ],
        out_specs=pl.BlockSpec((_BLK, D_OUT), lambda i: (i, 0)),
        out_shape=jax.ShapeDtypeStruct((NPAD, D_OUT), _f32),
    )(acc, den, b2)


def kernel(x, edge_index, W1, att_src1, att_dst1, bias1,
           W2, att_src2, att_dst2, bias2):
    loop = jnp.arange(N_NODES, dtype=_i32)
    src = jnp.concatenate([edge_index[0].astype(_i32), loop])
    dst = jnp.concatenate([edge_index[1].astype(_i32), loop])
    npad_e = EPAD - ETOT
    pad = jnp.arange(npad_e, dtype=_i32)
    src = jnp.concatenate([src, pad % N_NODES])
    dst = jnp.concatenate([dst, N_NODES + pad % (NPAD - N_NODES)])
    edges = jnp.stack([src.reshape(NW, CPW, CHUNK),
                       dst.reshape(NW, CPW, CHUNK)], axis=2)

    x_pad = jnp.pad(x, ((0, NPAD - N_NODES), (0, 0)))
    W2p = jnp.pad(W2, ((0, 0), (0, D_HID - D_OUT)))
    as2p = jnp.pad(att_src2, (0, D_HID - D_OUT)).reshape(D_HID, 1)
    ad2p = jnp.pad(att_dst2, (0, D_HID - D_OUT)).reshape(D_HID, 1)

    h1, as1, ad1 = _dense1(x_pad, W1,
                           att_src1.reshape(D_HID, 1),
                           att_dst1.reshape(D_HID, 1))
    acc1, den1 = _edge_pass1(edges, as1.reshape(NPAD), ad1.reshape(NPAD), h1)
    h2, as2, ad2 = _dense2(acc1, den1, bias1.reshape(1, D_HID), W2p,
                           as2p, ad2p)
    acc2, den2 = _edge_pass2(edges, as2.reshape(NPAD), ad2.reshape(NPAD), h2)
    out = _final(acc2, den2, bias2.reshape(1, D_OUT))
    return out[:N_NODES]


# speedup vs baseline: 51.2657x; 1.0558x over previous
import functools

import jax
import jax.numpy as jnp
from jax import lax
from jax.experimental import pallas as pl
from jax.experimental.pallas import tpu as pltpu
from jax.experimental.pallas import tpu_sc as plsc

N_NODES = 10000
NPAD = 10240
D_IN = 128
D_HID = 128
D_OUT = 64
N_EDGES = 320000
ETOT = N_EDGES + N_NODES
NW = 32
CHUNK = 128
CPW = 82
EPW = CPW * CHUNK
EPAD = EPW * NW
NT = 16
RPT = NPAD // NT

_f32 = jnp.float32
_i32 = jnp.int32


def _make_edge_pass(mult_width):
    D = D_HID
    mesh = plsc.VectorSubcoreMesh(core_axis_name="c", subcore_axis_name="s",
                                  num_cores=2, num_subcores=NT)

    @functools.partial(
        pl.kernel,
        out_type=(jax.ShapeDtypeStruct((2, NPAD, D), _f32),
                  jax.ShapeDtypeStruct((2, NPAD), _f32)),
        mesh=mesh,
        compiler_params=pltpu.CompilerParams(needs_layout_passes=False),
        scratch_types=[
            pltpu.VMEM_SHARED((NPAD, D), _f32),
            pltpu.VMEM_SHARED((NPAD,), _f32),
            pltpu.VMEM_SHARED((NPAD,), _f32),
            pltpu.VMEM_SHARED((NPAD,), _f32),
            [pltpu.VMEM((2, CHUNK), _i32)] * 2,
            [pltpu.VMEM((1, CHUNK), _i32)] * 2,
            [pltpu.VMEM((CHUNK,), _f32)] * 2,
            [pltpu.VMEM((CHUNK, D), _f32)] * 2,
            pltpu.VMEM((CHUNK,), _f32),
            pltpu.VMEM((CHUNK,), _f32),
            pltpu.VMEM((RPT,), _f32),
            [pltpu.SemaphoreType.DMA] * 2,
            [pltpu.SemaphoreType.DMA] * 2,
            [pltpu.SemaphoreType.DMA] * 2,
            [pltpu.SemaphoreType.DMA] * 2,
        ],
    )
    def edge_pass(edges_hbm, asrc_hbm, adst_hbm, h_hbm,
                  acc_out, den_out,
                  acc_sh, den_sh, asrc_sh, adst_sh,
                  idx, dstS, wv, rows, asg, adg, zvec,
                  isem, gsem, ssem, wsem):
        c = lax.axis_index("c")
        s = lax.axis_index("s")
        wid = c * NT + s
        base = s * RPT

        def zrow(r, carry):
            for k in range(D // 16):
                rows[0][r, pl.ds(k * 16, 16)] = jnp.zeros((16,), _f32)
            return carry
        lax.fori_loop(0, CHUNK, zrow, 0)

        def zv(m, carry):
            zvec[pl.ds(m * 16, 16)] = jnp.zeros((16,), _f32)
            return carry
        lax.fori_loop(0, RPT // 16, zv, 0)

        for t in range(RPT // CHUNK):
            pltpu.sync_copy(rows[0], acc_sh.at[pl.ds(base + t * CHUNK, CHUNK)])
        pltpu.sync_copy(zvec, den_sh.at[pl.ds(base, RPT)])

        @pl.when(s == 0)
        def _():
            pltpu.sync_copy(asrc_hbm, asrc_sh)

        @pl.when(s == 1)
        def _():
            pltpu.sync_copy(adst_hbm, adst_sh)

        pltpu.async_copy(edges_hbm.at[wid].at[0], idx[0], isem[0])
        plsc.subcore_barrier()

        def do_multiply_scatter(b1):
            @plsc.parallel_loop(0, CHUNK, step=1, unroll=4)
            def _(r):
                wb = plsc.load_gather(wv[b1], [jnp.broadcast_to(r, (16,))])
                for k in range(mult_width // 16):
                    rows[b1][r, pl.ds(k * 16, 16)] = (
                        rows[b1][r, pl.ds(k * 16, 16)] * wb)

            pltpu.async_copy(rows[b1], acc_sh.at[dstS[b1].at[0]], ssem[b1],
                             add=True)

        def body(t, b):
            b1 = 1 - b
            @pl.when(t >= 1)
            def _():
                pltpu.make_async_copy(wv[b1], den_sh.at[idx[b1].at[1]],
                                      wsem[b1]).wait()
                pltpu.make_async_copy(h_hbm.at[idx[b1].at[0]], rows[b1],
                                      gsem[b1]).wait()

            @pl.when(t + 1 < CPW)
            def _():
                pltpu.async_copy(edges_hbm.at[wid].at[t + 1], idx[b1],
                                 isem[b1])

            pltpu.make_async_copy(edges_hbm.at[wid].at[t], idx[b],
                                  isem[b]).wait()
            pltpu.sync_copy(asrc_sh.at[idx[b].at[0]], asg)
            pltpu.sync_copy(adst_sh.at[idx[b].at[1]], adg)
            for j in range(CHUNK // 16):
                e = asg[pl.ds(j * 16, 16)] + adg[pl.ds(j * 16, 16)]
                e = jnp.where(e >= 0.0, e, e * 0.2)
                wv[b][pl.ds(j * 16, 16)] = jnp.exp(e)
                dstS[b][0, pl.ds(j * 16, 16)] = idx[b][1, pl.ds(j * 16, 16)]
            pltpu.async_copy(wv[b], den_sh.at[idx[b].at[1]], wsem[b],
                             add=True)

            @pl.when(t >= 2)
            def _():
                pltpu.make_async_copy(rows[b], acc_sh.at[dstS[b].at[0]],
                                      ssem[b]).wait()
            pltpu.async_copy(h_hbm.at[idx[b].at[0]], rows[b], gsem[b])

            @pl.when(t >= 1)
            def _():
                do_multiply_scatter(b1)

        def gbody(g, carry):
            body(2 * g, 0)
            body(2 * g + 1, 1)
            return carry
        lax.fori_loop(0, CPW // 2, gbody, 0)

        pltpu.make_async_copy(h_hbm.at[idx[1].at[0]], rows[1], gsem[1]).wait()
        do_multiply_scatter(1)
        pltpu.make_async_copy(wv[1], den_sh.at[idx[1].at[1]], wsem[1]).wait()
        pltpu.make_async_copy(rows[0], acc_sh.at[dstS[0].at[0]],
                              ssem[0]).wait()
        pltpu.make_async_copy(rows[1], acc_sh.at[dstS[1].at[0]],
                              ssem[1]).wait()

        plsc.subcore_barrier()
        pltpu.sync_copy(acc_sh.at[pl.ds(base, RPT)],
                        acc_out.at[c].at[pl.ds(base, RPT)])
        pltpu.sync_copy(den_sh.at[pl.ds(base, RPT)],
                        den_out.at[c].at[pl.ds(base, RPT)])

    return edge_pass


_edge_pass1 = _make_edge_pass(D_HID)
_edge_pass2 = _make_edge_pass(D_OUT)


_BLK = 1024
_GRID = NPAD // _BLK


def _dense1_body(x_ref, w_ref, asc_ref, adc_ref, h_ref, as_ref, ad_ref):
    h = jnp.dot(x_ref[...], w_ref[...], preferred_element_type=_f32)
    h_ref[...] = h
    as_ref[...] = jnp.dot(h, asc_ref[...], preferred_element_type=_f32)
    ad_ref[...] = jnp.dot(h, adc_ref[...], preferred_element_type=_f32)


def _dense1(x_pad, W1, asc, adc):
    return pl.pallas_call(
        _dense1_body,
        grid=(_GRID,),
        in_specs=[
            pl.BlockSpec((_BLK, D_IN), lambda i: (i, 0)),
            pl.BlockSpec((D_IN, D_HID), lambda i: (0, 0)),
            pl.BlockSpec((D_HID, 1), lambda i: (0, 0)),
            pl.BlockSpec((D_HID, 1), lambda i: (0, 0)),
        ],
        out_specs=[
            pl.BlockSpec((_BLK, D_HID), lambda i: (i, 0)),
            pl.BlockSpec((_BLK, 1), lambda i: (i, 0)),
            pl.BlockSpec((_BLK, 1), lambda i: (i, 0)),
        ],
        out_shape=[
            jax.ShapeDtypeStruct((NPAD, D_HID), _f32),
            jax.ShapeDtypeStruct((NPAD, 1), _f32),
            jax.ShapeDtypeStruct((NPAD, 1), _f32),
        ],
    )(x_pad, W1, asc, adc)


def _dense2_body(acc_ref, den_ref, b_ref, w_ref, asc_ref, adc_ref,
                 h_ref, as_ref, ad_ref):
    den = den_ref[0] + den_ref[1]
    x2 = (acc_ref[0] + acc_ref[1]) / (den[:, None] + 1e-16) + b_ref[...]
    x2 = jnp.maximum(x2, 0.0)
    h2 = jnp.dot(x2, w_ref[...], preferred_element_type=_f32)
    h_ref[...] = h2
    as_ref[...] = jnp.dot(h2, asc_ref[...], preferred_element_type=_f32)
    ad_ref[...] = jnp.dot(h2, adc_ref[...], preferred_element_type=_f32)


def _dense2(acc, den, b1, W2, asc, adc):
    return pl.pallas_call(
        _dense2_body,
        grid=(_GRID,),
        in_specs=[
            pl.BlockSpec((2, _BLK, D_HID), lambda i: (0, i, 0)),
            pl.BlockSpec((2, _BLK), lambda i: (0, i)),
            pl.BlockSpec((1, D_HID), lambda i: (0, 0)),
            pl.BlockSpec((D_HID, D_HID), lambda i: (0, 0)),
            pl.BlockSpec((D_HID, 1), lambda i: (0, 0)),
            pl.BlockSpec((D_HID, 1), lambda i: (0, 0)),
        ],
        out_specs=[
            pl.BlockSpec((_BLK, D_HID), lambda i: (i, 0)),
            pl.BlockSpec((_BLK, 1), lambda i: (i, 0)),
            pl.BlockSpec((_BLK, 1), lambda i: (i, 0)),
        ],
        out_shape=[
            jax.ShapeDtypeStruct((NPAD, D_HID), _f32),
            jax.ShapeDtypeStruct((NPAD, 1), _f32),
            jax.ShapeDtypeStruct((NPAD, 1), _f32),
        ],
    )(acc, den, b1, W2, asc, adc)


def _final_body(acc_ref, den_ref, b_ref, out_ref):
    den = den_ref[0] + den_ref[1]
    o = (acc_ref[0, :, :D_OUT] + acc_ref[1, :, :D_OUT]) / (den[:, None] + 1e-16)
    o = o + b_ref[...]
    m = jnp.max(o, axis=1, keepdims=True)
    lse = jnp.log(jnp.sum(jnp.exp(o - m), axis=1, keepdims=True)) + m
    out_ref[...] = o - lse


def _final(acc, den, b2):
    return pl.pallas_call(
        _final_body,
        grid=(_GRID,),
        in_specs=[
            pl.BlockSpec((2, _BLK, D_HID), lambda i: (0, i, 0)),
            pl.BlockSpec((2, _BLK), lambda i: (0, i)),
            pl.BlockSpec((1, D_OUT), lambda i: (0, 0)),
        ],
        out_specs=pl.BlockSpec((_BLK, D_OUT), lambda i: (i, 0)),
        out_shape=jax.ShapeDtypeStruct((NPAD, D_OUT), _f32),
    )(acc, den, b2)


def kernel(x, edge_index, W1, att_src1, att_dst1, bias1,
           W2, att_src2, att_dst2, bias2):
    loop = jnp.arange(N_NODES, dtype=_i32)
    src = jnp.concatenate([edge_index[0].astype(_i32), loop])
    dst = jnp.concatenate([edge_index[1].astype(_i32), loop])
    npad_e = EPAD - ETOT
    pad = jnp.arange(npad_e, dtype=_i32)
    src = jnp.concatenate([src, pad % N_NODES])
    dst = jnp.concatenate([dst, N_NODES + pad % (NPAD - N_NODES)])
    edges = jnp.stack([src.reshape(NW, CPW, CHUNK),
                       dst.reshape(NW, CPW, CHUNK)], axis=2)

    x_pad = jnp.pad(x, ((0, NPAD - N_NODES), (0, 0)))
    W2p = jnp.pad(W2, ((0, 0), (0, D_HID - D_OUT)))
    as2p = jnp.pad(att_src2, (0, D_HID - D_OUT)).reshape(D_HID, 1)
    ad2p = jnp.pad(att_dst2, (0, D_HID - D_OUT)).reshape(D_HID, 1)

    h1, as1, ad1 = _dense1(x_pad, W1,
                           att_src1.reshape(D_HID, 1),
                           att_dst1.reshape(D_HID, 1))
    acc1, den1 = _edge_pass1(edges, as1.reshape(NPAD), ad1.reshape(NPAD), h1)
    h2, as2, ad2 = _dense2(acc1, den1, bias1.reshape(1, D_HID), W2p,
                           as2p, ad2p)
    acc2, den2 = _edge_pass2(edges, as2.reshape(NPAD), ad2.reshape(NPAD), h2)
    out = _final(acc2, den2, bias2.reshape(1, D_OUT))
    return out[:N_NODES]


# speedup vs baseline: 57.5504x; 1.1226x over previous
import functools

import jax
import jax.numpy as jnp
from jax import lax
from jax.experimental import pallas as pl
from jax.experimental.pallas import tpu as pltpu
from jax.experimental.pallas import tpu_sc as plsc

N_NODES = 10000
NPAD = 10240
D_IN = 128
D_HID = 128
D_OUT = 64
N_EDGES = 320000
ETOT = N_EDGES + N_NODES
NW = 32
CHUNK = 128
CPW = 82
EPW = CPW * CHUNK
EPAD = EPW * NW
NT = 16
RPT = NPAD // NT

_f32 = jnp.float32
_i32 = jnp.int32


def _make_edge_pass(mult_width):
    D = D_HID
    mesh = plsc.VectorSubcoreMesh(core_axis_name="c", subcore_axis_name="s",
                                  num_cores=2, num_subcores=NT)

    @functools.partial(
        pl.kernel,
        out_type=(jax.ShapeDtypeStruct((2, NPAD, D), _f32),
                  jax.ShapeDtypeStruct((2, NPAD), _f32)),
        mesh=mesh,
        compiler_params=pltpu.CompilerParams(needs_layout_passes=False),
        scratch_types=[
            pltpu.VMEM_SHARED((NPAD, D), _f32),
            pltpu.VMEM_SHARED((NPAD,), _f32),
            pltpu.VMEM_SHARED((NPAD,), _f32),
            pltpu.VMEM_SHARED((NPAD,), _f32),
            [pltpu.VMEM((2, CHUNK), _i32)] * 2,
            [pltpu.VMEM((1, CHUNK), _i32)] * 2,
            [pltpu.VMEM((CHUNK,), _f32)] * 2,
            [pltpu.VMEM((CHUNK, D), _f32)] * 2,
            pltpu.VMEM((CHUNK,), _f32),
            pltpu.VMEM((CHUNK,), _f32),
            pltpu.VMEM((RPT,), _f32),
            [pltpu.SemaphoreType.DMA] * 2,
            [pltpu.SemaphoreType.DMA] * 2,
            [pltpu.SemaphoreType.DMA] * 2,
            [pltpu.SemaphoreType.DMA] * 2,
        ],
    )
    def edge_pass(ei_hbm, asrc_hbm, adst_hbm, h_hbm,
                  acc_out, den_out,
                  acc_sh, den_sh, asrc_sh, adst_sh,
                  idx, dstS, wv, rows, asg, adg, zvec,
                  isem, gsem, ssem, wsem):
        c = lax.axis_index("c")
        s = lax.axis_index("s")
        wid = c * NT + s
        base = s * RPT

        def fetch_idx(t, b):
            off = jnp.minimum(wid * EPW + t * CHUNK, N_EDGES - CHUNK)
            pltpu.async_copy(ei_hbm.at[0].at[pl.ds(off, CHUNK)],
                             idx[b].at[0], isem[b])
            pltpu.async_copy(ei_hbm.at[1].at[pl.ds(off, CHUNK)],
                             idx[b].at[1], isem[b])

        def wait_idx(t, b):
            off = jnp.minimum(wid * EPW + t * CHUNK, N_EDGES - CHUNK)
            pltpu.make_async_copy(ei_hbm.at[0].at[pl.ds(off, CHUNK)],
                                  idx[b].at[0], isem[b]).wait()
            pltpu.make_async_copy(ei_hbm.at[1].at[pl.ds(off, CHUNK)],
                                  idx[b].at[1], isem[b]).wait()
            g = wid * EPW + t * CHUNK

            @pl.when(g >= N_EDGES)
            def _():
                for j in range(CHUNK // 16):
                    node = (g - N_EDGES + j * 16) + lax.iota(_i32, 16)
                    real = node < N_NODES
                    idx[b][0, pl.ds(j * 16, 16)] = jnp.where(
                        real, node, node & 63)
                    idx[b][1, pl.ds(j * 16, 16)] = jnp.where(
                        real, node, N_NODES + (node & 127))

        def zrow(r, carry):
            for k in range(D // 16):
                rows[0][r, pl.ds(k * 16, 16)] = jnp.zeros((16,), _f32)
            return carry
        lax.fori_loop(0, CHUNK, zrow, 0)

        def zv(m, carry):
            zvec[pl.ds(m * 16, 16)] = jnp.zeros((16,), _f32)
            return carry
        lax.fori_loop(0, RPT // 16, zv, 0)

        for t in range(RPT // CHUNK):
            pltpu.sync_copy(rows[0], acc_sh.at[pl.ds(base + t * CHUNK, CHUNK)])
        pltpu.sync_copy(zvec, den_sh.at[pl.ds(base, RPT)])

        @pl.when(s == 0)
        def _():
            pltpu.sync_copy(asrc_hbm, asrc_sh)

        @pl.when(s == 1)
        def _():
            pltpu.sync_copy(adst_hbm, adst_sh)

        fetch_idx(0, 0)
        plsc.subcore_barrier()

        def do_multiply_scatter(b1):
            @plsc.parallel_loop(0, CHUNK, step=1, unroll=4)
            def _(r):
                wb = plsc.load_gather(wv[b1], [jnp.broadcast_to(r, (16,))])
                for k in range(mult_width // 16):
                    rows[b1][r, pl.ds(k * 16, 16)] = (
                        rows[b1][r, pl.ds(k * 16, 16)] * wb)

            pltpu.async_copy(rows[b1], acc_sh.at[dstS[b1].at[0]], ssem[b1],
                             add=True)

        def body(t, b):
            b1 = 1 - b
            @pl.when(t >= 1)
            def _():
                pltpu.make_async_copy(wv[b1], den_sh.at[idx[b1].at[1]],
                                      wsem[b1]).wait()
                pltpu.make_async_copy(h_hbm.at[idx[b1].at[0]], rows[b1],
                                      gsem[b1]).wait()

            @pl.when(t + 1 < CPW)
            def _():
                fetch_idx(t + 1, b1)

            wait_idx(t, b)
            pltpu.sync_copy(asrc_sh.at[idx[b].at[0]], asg)
            pltpu.sync_copy(adst_sh.at[idx[b].at[1]], adg)
            for j in range(CHUNK // 16):
                e = asg[pl.ds(j * 16, 16)] + adg[pl.ds(j * 16, 16)]
                e = jnp.where(e >= 0.0, e, e * 0.2)
                wv[b][pl.ds(j * 16, 16)] = jnp.exp(e)
                dstS[b][0, pl.ds(j * 16, 16)] = idx[b][1, pl.ds(j * 16, 16)]
            pltpu.async_copy(wv[b], den_sh.at[idx[b].at[1]], wsem[b],
                             add=True)

            @pl.when(t >= 2)
            def _():
                pltpu.make_async_copy(rows[b], acc_sh.at[dstS[b].at[0]],
                                      ssem[b]).wait()
            pltpu.async_copy(h_hbm.at[idx[b].at[0]], rows[b], gsem[b])

            @pl.when(t >= 1)
            def _():
                do_multiply_scatter(b1)

        def gbody(g, carry):
            body(2 * g, 0)
            body(2 * g + 1, 1)
            return carry
        lax.fori_loop(0, CPW // 2, gbody, 0)

        pltpu.make_async_copy(h_hbm.at[idx[1].at[0]], rows[1], gsem[1]).wait()
        do_multiply_scatter(1)
        pltpu.make_async_copy(wv[1], den_sh.at[idx[1].at[1]], wsem[1]).wait()
        pltpu.make_async_copy(rows[0], acc_sh.at[dstS[0].at[0]],
                              ssem[0]).wait()
        pltpu.make_async_copy(rows[1], acc_sh.at[dstS[1].at[0]],
                              ssem[1]).wait()

        plsc.subcore_barrier()
        pltpu.sync_copy(acc_sh.at[pl.ds(base, RPT)],
                        acc_out.at[c].at[pl.ds(base, RPT)])
        pltpu.sync_copy(den_sh.at[pl.ds(base, RPT)],
                        den_out.at[c].at[pl.ds(base, RPT)])

    return edge_pass


_edge_pass1 = _make_edge_pass(D_HID)
_edge_pass2 = _make_edge_pass(D_OUT)


_BLK = 1024
_GRID = NPAD // _BLK


def _dense1_body(x_ref, w_ref, asc_ref, adc_ref, h_ref, as_ref, ad_ref):
    h = jnp.dot(x_ref[...], w_ref[...], preferred_element_type=_f32)
    h_ref[...] = h
    as_ref[...] = jnp.sum(h * asc_ref[...], axis=1)
    ad_ref[...] = jnp.sum(h * adc_ref[...], axis=1)


def _dense1(x_pad, W1, asc, adc):
    return pl.pallas_call(
        _dense1_body,
        grid=(_GRID,),
        in_specs=[
            pl.BlockSpec((_BLK, D_IN), lambda i: (i, 0)),
            pl.BlockSpec((D_IN, D_HID), lambda i: (0, 0)),
            pl.BlockSpec((1, D_HID), lambda i: (0, 0)),
            pl.BlockSpec((1, D_HID), lambda i: (0, 0)),
        ],
        out_specs=[
            pl.BlockSpec((_BLK, D_HID), lambda i: (i, 0)),
            pl.BlockSpec((_BLK,), lambda i: (i,)),
            pl.BlockSpec((_BLK,), lambda i: (i,)),
        ],
        out_shape=[
            jax.ShapeDtypeStruct((NPAD, D_HID), _f32),
            jax.ShapeDtypeStruct((NPAD,), _f32),
            jax.ShapeDtypeStruct((NPAD,), _f32),
        ],
    )(x_pad, W1, asc, adc)


def _dense2_body(acc_ref, den_ref, b_ref, w_ref, asc_ref, adc_ref,
                 h_ref, as_ref, ad_ref):
    den = den_ref[0] + den_ref[1]
    x2 = (acc_ref[0] + acc_ref[1]) / (den[:, None] + 1e-16) + b_ref[...]
    x2 = jnp.maximum(x2, 0.0)
    h2 = jnp.dot(x2, w_ref[...], preferred_element_type=_f32)
    h_ref[...] = h2
    as_ref[...] = jnp.sum(h2 * asc_ref[...], axis=1)
    ad_ref[...] = jnp.sum(h2 * adc_ref[...], axis=1)


def _dense2(acc, den, b1, W2, asc, adc):
    return pl.pallas_call(
        _dense2_body,
        grid=(_GRID,),
        in_specs=[
            pl.BlockSpec((2, _BLK, D_HID), lambda i: (0, i, 0)),
            pl.BlockSpec((2, _BLK), lambda i: (0, i)),
            pl.BlockSpec((1, D_HID), lambda i: (0, 0)),
            pl.BlockSpec((D_HID, D_HID), lambda i: (0, 0)),
            pl.BlockSpec((1, D_HID), lambda i: (0, 0)),
            pl.BlockSpec((1, D_HID), lambda i: (0, 0)),
        ],
        out_specs=[
            pl.BlockSpec((_BLK, D_HID), lambda i: (i, 0)),
            pl.BlockSpec((_BLK,), lambda i: (i,)),
            pl.BlockSpec((_BLK,), lambda i: (i,)),
        ],
        out_shape=[
            jax.ShapeDtypeStruct((NPAD, D_HID), _f32),
            jax.ShapeDtypeStruct((NPAD,), _f32),
            jax.ShapeDtypeStruct((NPAD,), _f32),
        ],
    )(acc, den, b1, W2, asc, adc)


def _final_body(acc_ref, den_ref, b_ref, out_ref):
    den = den_ref[0] + den_ref[1]
    o = (acc_ref[0, :, :D_OUT] + acc_ref[1, :, :D_OUT]) / (den[:, None] + 1e-16)
    o = o + b_ref[...]
    m = jnp.max(o, axis=1, keepdims=True)
    lse = jnp.log(jnp.sum(jnp.exp(o - m), axis=1, keepdims=True)) + m
    out_ref[...] = o - lse


def _final(acc, den, b2):
    return pl.pallas_call(
        _final_body,
        grid=(_GRID,),
        in_specs=[
            pl.BlockSpec((2, _BLK, D_HID), lambda i: (0, i, 0)),
            pl.BlockSpec((2, _BLK), lambda i: (0, i)),
            pl.BlockSpec((1, D_OUT), lambda i: (0, 0)),
        ],
        out_specs=pl.BlockSpec((_BLK, D_OUT), lambda i: (i, 0)),
        out_shape=jax.ShapeDtypeStruct((N_NODES, D_OUT), _f32),
    )(acc, den, b2)


def kernel(x, edge_index, W1, att_src1, att_dst1, bias1,
           W2, att_src2, att_dst2, bias2):
    ei = edge_index.astype(_i32)

    x_pad = jnp.pad(x, ((0, NPAD - N_NODES), (0, 0)))
    W2p = jnp.pad(W2, ((0, 0), (0, D_HID - D_OUT)))
    as2p = jnp.pad(att_src2, (0, D_HID - D_OUT)).reshape(1, D_HID)
    ad2p = jnp.pad(att_dst2, (0, D_HID - D_OUT)).reshape(1, D_HID)

    h1, as1, ad1 = _dense1(x_pad, W1,
                           att_src1.reshape(1, D_HID),
                           att_dst1.reshape(1, D_HID))
    acc1, den1 = _edge_pass1(ei, as1, ad1, h1)
    h2, as2, ad2 = _dense2(acc1, den1, bias1.reshape(1, D_HID), W2p,
                           as2p, ad2p)
    acc2, den2 = _edge_pass2(ei, as2, ad2, h2)
    return _final(acc2, den2, bias2.reshape(1, D_OUT))


# speedup vs baseline: 67.0984x; 1.1659x over previous
import functools

import jax
import jax.numpy as jnp
from jax import lax
from jax.experimental import pallas as pl
from jax.experimental.pallas import tpu as pltpu
from jax.experimental.pallas import tpu_sc as plsc

N_NODES = 10000
NPAD = 10240
D_IN = 128
D_HID = 128
D_OUT = 64
N_EDGES = 320000
ETOT = N_EDGES + N_NODES
NW = 32
CHUNK = 128
CPW = 84
EPW = CPW * CHUNK
EPAD = EPW * NW
NT = 16
RPT = NPAD // NT

_f32 = jnp.float32
_i32 = jnp.int32


def _make_edge_pass(mult_width):
    D = D_HID
    mesh = plsc.VectorSubcoreMesh(core_axis_name="c", subcore_axis_name="s",
                                  num_cores=2, num_subcores=NT)

    @functools.partial(
        pl.kernel,
        out_type=(jax.ShapeDtypeStruct((2, NPAD, D), _f32),
                  jax.ShapeDtypeStruct((2, NPAD), _f32)),
        mesh=mesh,
        compiler_params=pltpu.CompilerParams(needs_layout_passes=False),
        scratch_types=[
            pltpu.VMEM_SHARED((NPAD, D), _f32),
            pltpu.VMEM_SHARED((NPAD,), _f32),
            pltpu.VMEM_SHARED((NPAD,), _f32),
            pltpu.VMEM_SHARED((NPAD,), _f32),
            [pltpu.VMEM((2, CHUNK), _i32)] * 4,
            [pltpu.VMEM((1, CHUNK), _i32)] * 2,
            [pltpu.VMEM((CHUNK,), _f32)] * 2,
            [pltpu.VMEM((CHUNK, D), _f32)] * 2,
            [pltpu.VMEM((CHUNK,), _f32)] * 2,
            [pltpu.VMEM((CHUNK,), _f32)] * 2,
            pltpu.VMEM((RPT,), _f32),
            [pltpu.SemaphoreType.DMA] * 4,
            [pltpu.SemaphoreType.DMA] * 2,
            [pltpu.SemaphoreType.DMA] * 2,
            [pltpu.SemaphoreType.DMA] * 2,
            [pltpu.SemaphoreType.DMA] * 2,
        ],
    )
    def edge_pass(ei_hbm, asrc_hbm, adst_hbm, h_hbm,
                  acc_out, den_out,
                  acc_sh, den_sh, asrc_sh, adst_sh,
                  idx, dstS, wv, rows, asg, adg, zvec,
                  isem, asem, gsem, ssem, wsem):
        c = lax.axis_index("c")
        s = lax.axis_index("s")
        wid = c * NT + s
        base = s * RPT

        def fetch_idx(t, b):
            off = jnp.minimum(wid * EPW + t * CHUNK, N_EDGES - CHUNK)
            pltpu.async_copy(ei_hbm.at[0].at[pl.ds(off, CHUNK)],
                             idx[b].at[0], isem[b])
            pltpu.async_copy(ei_hbm.at[1].at[pl.ds(off, CHUNK)],
                             idx[b].at[1], isem[b])

        def wait_idx(t, b):
            off = jnp.minimum(wid * EPW + t * CHUNK, N_EDGES - CHUNK)
            pltpu.make_async_copy(ei_hbm.at[0].at[pl.ds(off, CHUNK)],
                                  idx[b].at[0], isem[b]).wait()
            pltpu.make_async_copy(ei_hbm.at[1].at[pl.ds(off, CHUNK)],
                                  idx[b].at[1], isem[b]).wait()
            g = wid * EPW + t * CHUNK

            @pl.when(g >= N_EDGES)
            def _():
                for j in range(CHUNK // 16):
                    node = (g - N_EDGES + j * 16) + lax.iota(_i32, 16)
                    real = node < N_NODES
                    idx[b][0, pl.ds(j * 16, 16)] = jnp.where(
                        real, node, node & 63)
                    idx[b][1, pl.ds(j * 16, 16)] = jnp.where(
                        real, node, N_NODES + (node & 127))

        def fetch_a(bi, ba):
            pltpu.async_copy(asrc_sh.at[idx[bi].at[0]], asg[ba], asem[ba])
            pltpu.async_copy(adst_sh.at[idx[bi].at[1]], adg[ba], asem[ba])

        def wait_a(bi, ba):
            pltpu.make_async_copy(asrc_sh.at[idx[bi].at[0]], asg[ba],
                                  asem[ba]).wait()
            pltpu.make_async_copy(adst_sh.at[idx[bi].at[1]], adg[ba],
                                  asem[ba]).wait()

        def zrow(r, carry):
            for k in range(D // 16):
                rows[0][r, pl.ds(k * 16, 16)] = jnp.zeros((16,), _f32)
            return carry
        lax.fori_loop(0, CHUNK, zrow, 0)

        def zv(m, carry):
            zvec[pl.ds(m * 16, 16)] = jnp.zeros((16,), _f32)
            return carry
        lax.fori_loop(0, RPT // 16, zv, 0)

        for t in range(RPT // CHUNK):
            pltpu.sync_copy(rows[0], acc_sh.at[pl.ds(base + t * CHUNK, CHUNK)])
        pltpu.sync_copy(zvec, den_sh.at[pl.ds(base, RPT)])

        @pl.when(s == 0)
        def _():
            pltpu.sync_copy(asrc_hbm, asrc_sh)

        @pl.when(s == 1)
        def _():
            pltpu.sync_copy(adst_hbm, adst_sh)

        fetch_idx(0, 0)
        fetch_idx(1, 1)
        plsc.subcore_barrier()
        wait_idx(0, 0)
        fetch_a(0, 0)

        def do_multiply_scatter(b1):
            @plsc.parallel_loop(0, CHUNK, step=1, unroll=4)
            def _(r):
                wb = plsc.load_gather(wv[b1], [jnp.broadcast_to(r, (16,))])
                for k in range(mult_width // 16):
                    rows[b1][r, pl.ds(k * 16, 16)] = (
                        rows[b1][r, pl.ds(k * 16, 16)] * wb)

            pltpu.async_copy(rows[b1], acc_sh.at[dstS[b1].at[0]], ssem[b1],
                             add=True)

        def body(t, bi, b):
            b1 = 1 - b
            bi1 = (bi + 1) % 4
            bi2 = (bi + 2) % 4

            @pl.when(t >= 2)
            def _():
                pltpu.make_async_copy(wv[b], den_sh.at[idx[bi2].at[1]],
                                      wsem[b]).wait()

            @pl.when(t + 2 < CPW)
            def _():
                fetch_idx(t + 2, bi2)

            @pl.when(t + 1 < CPW)
            def _():
                wait_idx(t + 1, bi1)
                fetch_a(bi1, b1)

            wait_a(bi, b)

            @pl.when(t >= 2)
            def _():
                pltpu.make_async_copy(rows[b], acc_sh.at[dstS[b].at[0]],
                                      ssem[b]).wait()
            for j in range(CHUNK // 16):
                e = asg[b][pl.ds(j * 16, 16)] + adg[b][pl.ds(j * 16, 16)]
                e = jnp.where(e >= 0.0, e, e * 0.2)
                wv[b][pl.ds(j * 16, 16)] = jnp.exp(e)
                dstS[b][0, pl.ds(j * 16, 16)] = idx[bi][1, pl.ds(j * 16, 16)]
            pltpu.async_copy(wv[b], den_sh.at[idx[bi].at[1]], wsem[b],
                             add=True)
            pltpu.async_copy(h_hbm.at[idx[bi].at[0]], rows[b], gsem[b])

            @pl.when(t >= 1)
            def _():
                pltpu.make_async_copy(h_hbm.at[idx[(bi - 1) % 4].at[0]],
                                      rows[b1], gsem[b1]).wait()
                do_multiply_scatter(b1)

        def gbody(g, carry):
            for q in range(4):
                body(4 * g + q, q, q % 2)
            return carry
        lax.fori_loop(0, CPW // 4, gbody, 0)

        bL = (CPW - 1) % 2
        pltpu.make_async_copy(h_hbm.at[idx[(CPW - 1) % 4].at[0]], rows[bL],
                              gsem[bL]).wait()
        do_multiply_scatter(bL)
        pltpu.make_async_copy(wv[0], den_sh.at[idx[0].at[1]], wsem[0]).wait()
        pltpu.make_async_copy(wv[1], den_sh.at[idx[1].at[1]], wsem[1]).wait()
        pltpu.make_async_copy(rows[0], acc_sh.at[dstS[0].at[0]],
                              ssem[0]).wait()
        pltpu.make_async_copy(rows[1], acc_sh.at[dstS[1].at[0]],
                              ssem[1]).wait()

        plsc.subcore_barrier()
        pltpu.sync_copy(acc_sh.at[pl.ds(base, RPT)],
                        acc_out.at[c].at[pl.ds(base, RPT)])
        pltpu.sync_copy(den_sh.at[pl.ds(base, RPT)],
                        den_out.at[c].at[pl.ds(base, RPT)])

    return edge_pass


_edge_pass1 = _make_edge_pass(D_HID)
_edge_pass2 = _make_edge_pass(D_OUT)


_BLK = 1024
_GRID = NPAD // _BLK


def _dense1_body(x_ref, w_ref, asc_ref, adc_ref, h_ref, as_ref, ad_ref):
    h = jnp.dot(x_ref[...], w_ref[...], preferred_element_type=_f32)
    h_ref[...] = h
    as_ref[...] = jnp.sum(h * asc_ref[...], axis=1)
    ad_ref[...] = jnp.sum(h * adc_ref[...], axis=1)


def _dense1(x_pad, W1, asc, adc):
    return pl.pallas_call(
        _dense1_body,
        grid=(_GRID,),
        in_specs=[
            pl.BlockSpec((_BLK, D_IN), lambda i: (i, 0)),
            pl.BlockSpec((D_IN, D_HID), lambda i: (0, 0)),
            pl.BlockSpec((1, D_HID), lambda i: (0, 0)),
            pl.BlockSpec((1, D_HID), lambda i: (0, 0)),
        ],
        out_specs=[
            pl.BlockSpec((_BLK, D_HID), lambda i: (i, 0)),
            pl.BlockSpec((_BLK,), lambda i: (i,)),
            pl.BlockSpec((_BLK,), lambda i: (i,)),
        ],
        out_shape=[
            jax.ShapeDtypeStruct((NPAD, D_HID), _f32),
            jax.ShapeDtypeStruct((NPAD,), _f32),
            jax.ShapeDtypeStruct((NPAD,), _f32),
        ],
    )(x_pad, W1, asc, adc)


def _dense2_body(acc_ref, den_ref, b_ref, w_ref, asc_ref, adc_ref,
                 h_ref, as_ref, ad_ref):
    den = den_ref[0] + den_ref[1]
    x2 = (acc_ref[0] + acc_ref[1]) / (den[:, None] + 1e-16) + b_ref[...]
    x2 = jnp.maximum(x2, 0.0)
    h2 = jnp.dot(x2, w_ref[...], preferred_element_type=_f32)
    h_ref[...] = h2
    as_ref[...] = jnp.sum(h2 * asc_ref[...], axis=1)
    ad_ref[...] = jnp.sum(h2 * adc_ref[...], axis=1)


def _dense2(acc, den, b1, W2, asc, adc):
    return pl.pallas_call(
        _dense2_body,
        grid=(_GRID,),
        in_specs=[
            pl.BlockSpec((2, _BLK, D_HID), lambda i: (0, i, 0)),
            pl.BlockSpec((2, _BLK), lambda i: (0, i)),
            pl.BlockSpec((1, D_HID), lambda i: (0, 0)),
            pl.BlockSpec((D_HID, D_HID), lambda i: (0, 0)),
            pl.BlockSpec((1, D_HID), lambda i: (0, 0)),
            pl.BlockSpec((1, D_HID), lambda i: (0, 0)),
        ],
        out_specs=[
            pl.BlockSpec((_BLK, D_HID), lambda i: (i, 0)),
            pl.BlockSpec((_BLK,), lambda i: (i,)),
            pl.BlockSpec((_BLK,), lambda i: (i,)),
        ],
        out_shape=[
            jax.ShapeDtypeStruct((NPAD, D_HID), _f32),
            jax.ShapeDtypeStruct((NPAD,), _f32),
            jax.ShapeDtypeStruct((NPAD,), _f32),
        ],
    )(acc, den, b1, W2, asc, adc)


def _final_body(acc_ref, den_ref, b_ref, out_ref):
    den = den_ref[0] + den_ref[1]
    o = (acc_ref[0, :, :D_OUT] + acc_ref[1, :, :D_OUT]) / (den[:, None] + 1e-16)
    o = o + b_ref[...]
    m = jnp.max(o, axis=1, keepdims=True)
    lse = jnp.log(jnp.sum(jnp.exp(o - m), axis=1, keepdims=True)) + m
    out_ref[...] = o - lse


def _final(acc, den, b2):
    return pl.pallas_call(
        _final_body,
        grid=(_GRID,),
        in_specs=[
            pl.BlockSpec((2, _BLK, D_HID), lambda i: (0, i, 0)),
            pl.BlockSpec((2, _BLK), lambda i: (0, i)),
            pl.BlockSpec((1, D_OUT), lambda i: (0, 0)),
        ],
        out_specs=pl.BlockSpec((_BLK, D_OUT), lambda i: (i, 0)),
        out_shape=jax.ShapeDtypeStruct((N_NODES, D_OUT), _f32),
    )(acc, den, b2)


def kernel(x, edge_index, W1, att_src1, att_dst1, bias1,
           W2, att_src2, att_dst2, bias2):
    ei = edge_index.astype(_i32)

    W2p = jnp.pad(W2, ((0, 0), (0, D_HID - D_OUT)))
    as2p = jnp.pad(att_src2, (0, D_HID - D_OUT)).reshape(1, D_HID)
    ad2p = jnp.pad(att_dst2, (0, D_HID - D_OUT)).reshape(1, D_HID)

    h1, as1, ad1 = _dense1(x, W1,
                           att_src1.reshape(1, D_HID),
                           att_dst1.reshape(1, D_HID))
    acc1, den1 = _edge_pass1(ei, as1, ad1, h1)
    h2, as2, ad2 = _dense2(acc1, den1, bias1.reshape(1, D_HID), W2p,
                           as2p, ad2p)
    acc2, den2 = _edge_pass2(ei, as2, ad2, h2)
    return _final(acc2, den2, bias2.reshape(1, D_OUT))
